# baseline (speedup 1.0000x reference)
import functools

import jax
import jax.numpy as jnp
from jax import lax
from jax.experimental import pallas as pl
from jax.experimental.pallas import tpu as pltpu
from jax.experimental.pallas import tpu_sc as plsc

NUM_ITERATIONS = 2
GAMMA = 1.0

_NC = 2
_NS = 16
_NW = _NC * _NS
_L = 16

_EDGE_BLOCK = 2000
_ACC_BLOCK = 4000


def _npad(n_nodes):
    return ((n_nodes + _NW * 4 - 1) // (_NW * 4)) * (_NW * 4)


def _sc_mesh():
    return plsc.VectorSubcoreMesh(core_axis_name="c", subcore_axis_name="s",
                                  num_cores=_NC, num_subcores=_NS)


@functools.lru_cache(maxsize=None)
def _build_edge_update(n_edges, n_nodes):
    epw = n_edges // _NW
    block = _EDGE_BLOCK
    nblocks = epw // block

    def body(src_hbm, dst_hbm, sn_hbm, se_hbm, gm_hbm, snew_hbm,
             table_v, gm_v,
             src0_v, dst0_v, sin0_v, sout0_v, sem0,
             src1_v, dst1_v, sin1_v, sout1_v, sem1):
        wid = lax.axis_index("c") * _NS + lax.axis_index("s")
        pltpu.sync_copy(sn_hbm, table_v)
        pltpu.sync_copy(gm_hbm, gm_v)
        base0 = wid * epw
        slots = ((src0_v, dst0_v, sin0_v, sout0_v, sem0),
                 (src1_v, dst1_v, sin1_v, sout1_v, sem1))

        def copies(b, slot):
            src_v, dst_v, sin_v, _, sem = slot
            base = base0 + b * block
            return (
                pltpu.make_async_copy(src_hbm.at[pl.ds(base, block)],
                                      src_v, sem),
                pltpu.make_async_copy(dst_hbm.at[pl.ds(base, block)],
                                      dst_v, sem),
                pltpu.make_async_copy(se_hbm.at[pl.ds(base, block)],
                                      sin_v, sem))

        def start_in(b, slot):
            for c in copies(b, slot):
                c.start()

        def wait_in(b, slot):
            for c in copies(b, slot):
                c.wait()

        def compute(b, slot):
            src_v, dst_v, sin_v, sout_v, _ = slot
            gm = gm_v[...]

            def step(i, c2):
                o = i * _L
                isrc = src_v[pl.ds(o, _L)]
                idst = dst_v[pl.ds(o, _L)]
                nap = jnp.maximum(plsc.load_gather(table_v, [isrc]),
                                  plsc.load_gather(table_v, [idst]))
                f = 1.0 + nap * gm
                s = sin_v[pl.ds(o, _L)]
                sf = s * f
                sout_v[pl.ds(o, _L)] = sf / (1.0 - s + sf)
                return c2
            lax.fori_loop(0, block // _L, step, 0, unroll=4)
            base = base0 + b * block
            pltpu.sync_copy(sout_v, snew_hbm.at[pl.ds(base, block)])

        start_in(0, slots[0])

        def pair(g, carry):
            b0 = 2 * g
            start_in(b0 + 1, slots[1])
            wait_in(b0, slots[0])
            compute(b0, slots[0])

            @pl.when(g + 1 < nblocks // 2)
            def _():
                start_in(b0 + 2, slots[0])
            wait_in(b0 + 1, slots[1])
            compute(b0 + 1, slots[1])
            return carry
        lax.fori_loop(0, nblocks // 2, pair, 0)

    buf = lambda dt: pltpu.VMEM((block,), dt)
    return pl.kernel(
        body,
        out_type=jax.ShapeDtypeStruct((n_edges,), jnp.float32),
        mesh=_sc_mesh(),
        scratch_types=(pltpu.VMEM((n_nodes,), jnp.float32),
                       pltpu.VMEM((_L,), jnp.float32),
                       buf(jnp.int32), buf(jnp.int32), buf(jnp.float32),
                       buf(jnp.float32), pltpu.SemaphoreType.DMA,
                       buf(jnp.int32), buf(jnp.int32), buf(jnp.float32),
                       buf(jnp.float32), pltpu.SemaphoreType.DMA),
        compiler_params=pltpu.CompilerParams(needs_layout_passes=False))


@functools.lru_cache(maxsize=None)
def _build_edge_accum(n_edges, n_nodes, with_vals):
    npad = _npad(n_nodes)
    epw = n_edges // _NW
    block = _ACC_BLOCK
    nblocks = epw // block

    nbuf = [pltpu.VMEM((block,), jnp.int32),
            pltpu.VMEM((block,), jnp.int32)]
    if with_vals:
        nbuf.append(pltpu.VMEM((block,), jnp.float32))
    nbuf.append(pltpu.SemaphoreType.DMA)
    scratch = [pltpu.VMEM((npad,), jnp.float32)] + nbuf + nbuf

    def body(src_hbm, dst_hbm, *rest):
        nslot = 4 if with_vals else 3
        if with_vals:
            vals_hbm, out_hbm = rest[:2]
            rest = rest[2:]
        else:
            out_hbm = rest[0]
            rest = rest[1:]
        acc_v = rest[0]
        slots = (rest[1:1 + nslot], rest[1 + nslot:1 + 2 * nslot])
        wid = lax.axis_index("c") * _NS + lax.axis_index("s")

        def zstep(i, carry):
            acc_v[pl.ds(i * _L, _L)] = jnp.zeros((_L,), jnp.float32)
            return carry
        lax.fori_loop(0, npad // _L, zstep, 0, unroll=4)

        base0 = wid * epw
        ones = jnp.ones((_L,), jnp.float32)

        def copies(b, slot):
            base = base0 + b * block
            sem = slot[-1]
            cps = [pltpu.make_async_copy(src_hbm.at[pl.ds(base, block)],
                                         slot[0], sem),
                   pltpu.make_async_copy(dst_hbm.at[pl.ds(base, block)],
                                         slot[1], sem)]
            if with_vals:
                cps.append(pltpu.make_async_copy(
                    vals_hbm.at[pl.ds(base, block)], slot[2], sem))
            return cps

        def start_in(b, slot):
            for c in copies(b, slot):
                c.start()

        def wait_in(b, slot):
            for c in copies(b, slot):
                c.wait()

        def compute(slot):
            src_v, dst_v = slot[0], slot[1]

            def step(i, c2):
                o = i * _L
                v = slot[2][pl.ds(o, _L)] if with_vals else ones
                plsc.addupdate_scatter(acc_v, [src_v[pl.ds(o, _L)]], v)
                plsc.addupdate_scatter(acc_v, [dst_v[pl.ds(o, _L)]], v)
                return c2
            lax.fori_loop(0, block // _L, step, 0, unroll=4)

        start_in(0, slots[0])

        def pair(g, carry):
            b0 = 2 * g
            start_in(b0 + 1, slots[1])
            wait_in(b0, slots[0])
            compute(slots[0])

            @pl.when(g + 1 < nblocks // 2)
            def _():
                start_in(b0 + 2, slots[0])
            wait_in(b0 + 1, slots[1])
            compute(slots[1])
            return carry
        lax.fori_loop(0, nblocks // 2, pair, 0)

        pltpu.sync_copy(acc_v, out_hbm.at[wid])

    return pl.kernel(
        body,
        out_type=jax.ShapeDtypeStruct((_NW, npad), jnp.float32),
        mesh=_sc_mesh(),
        scratch_types=tuple(scratch),
        compiler_params=pltpu.CompilerParams(needs_layout_passes=False))


_EDGE_COLS = 51200
_NODE_COLS = 16384


def _abnormal_t_body(x_ref, o_ref):
    x = x_ref[...]
    m = jnp.max(x, axis=0)
    e = jnp.exp(x - m[None, :])
    z = jnp.sum(e, axis=0)
    o_ref[...] = 1.0 - e[0, :] / z


@functools.lru_cache(maxsize=None)
def _build_abnormal(cols, n_classes, block_cols):
    grid = pl.cdiv(cols, block_cols)
    return pl.pallas_call(
        _abnormal_t_body,
        grid=(grid,),
        in_specs=[pl.BlockSpec((n_classes, block_cols), lambda i: (0, i))],
        out_specs=pl.BlockSpec((block_cols,), lambda i: (i,)),
        out_shape=jax.ShapeDtypeStruct((cols,), jnp.float32),
    )


@functools.lru_cache(maxsize=None)
def _build_node_update(n_nodes, first):
    npad = _npad(n_nodes)

    def body(s_ref, sums_ref, deg_ref, gm_ref, snew_ref, *rest):
        s = s_ref[...]
        sums = jnp.sum(sums_ref[...], axis=0)[:n_nodes]
        if first:
            deg = jnp.sum(deg_ref[...], axis=0)[:n_nodes]
        else:
            deg = deg_ref[...]
        m = sums / (deg + 1e-6)
        f = 1.0 + gm_ref[0] * m
        sf = s * f
        snew_ref[...] = sf / (1.0 - s + sf)
        if first:
            rest[0][...] = deg

    deg_spec = (pl.BlockSpec((_NW, npad), lambda: (0, 0)) if first
                else pl.BlockSpec((n_nodes,), lambda: (0,)))
    if first:
        out_shape = (jax.ShapeDtypeStruct((n_nodes,), jnp.float32),
                     jax.ShapeDtypeStruct((n_nodes,), jnp.float32))
        out_specs = (pl.BlockSpec((n_nodes,), lambda: (0,)),
                     pl.BlockSpec((n_nodes,), lambda: (0,)))
    else:
        out_shape = jax.ShapeDtypeStruct((n_nodes,), jnp.float32)
        out_specs = pl.BlockSpec((n_nodes,), lambda: (0,))

    return pl.pallas_call(
        body,
        grid=(),
        in_specs=[pl.BlockSpec((n_nodes,), lambda: (0,)),
                  pl.BlockSpec((_NW, npad), lambda: (0, 0)),
                  deg_spec,
                  pl.BlockSpec(memory_space=pltpu.SMEM)],
        out_specs=out_specs,
        out_shape=out_shape,
    )


def _refine_t_body(x_ref, s0_ref, s2_ref, o_ref):
    x = x_ref[...]
    m = jnp.max(x, axis=0, keepdims=True)
    e = jnp.exp(x - m)
    z = jnp.sum(e, axis=0, keepdims=True)
    p = e / z
    s0 = s0_ref[...]
    s2 = s2_ref[...]
    r = (s2 / jnp.maximum(s0, 1e-30))[None, :]
    row = lax.broadcasted_iota(jnp.int32, x.shape, 0)
    vals = jnp.where(row == 0, (1.0 - s2)[None, :], p * r)
    o_ref[...] = jnp.log(vals + 1e-9)


@functools.lru_cache(maxsize=None)
def _build_refine(cols, n_classes, block_cols):
    grid = pl.cdiv(cols, block_cols)
    return pl.pallas_call(
        _refine_t_body,
        grid=(grid,),
        in_specs=[pl.BlockSpec((n_classes, block_cols), lambda i: (0, i)),
                  pl.BlockSpec((block_cols,), lambda i: (i,)),
                  pl.BlockSpec((block_cols,), lambda i: (i,))],
        out_specs=pl.BlockSpec((n_classes, block_cols), lambda i: (0, i)),
        out_shape=jax.ShapeDtypeStruct((n_classes, cols), jnp.float32),
    )


def kernel(node_logits, edge_logits, edge_index, node_factor_weights,
           edge_factor_weights):
    n_nodes, node_classes = node_logits.shape
    n_edges, edge_classes = edge_logits.shape

    src = edge_index[0].astype(jnp.int32)
    dst = edge_index[1].astype(jnp.int32)

    gm_e = GAMMA * jnp.mean(edge_factor_weights[1:, 1:].astype(jnp.float32))
    gm_n = GAMMA * jnp.mean(node_factor_weights[1:, 1:].astype(jnp.float32))
    gm_e_vec = jnp.full((_L,), gm_e, jnp.float32)
    gm_n_s = jnp.reshape(gm_n, (1,))

    edge_t = jnp.transpose(edge_logits)
    node_t = jnp.transpose(node_logits)
    s_e = _build_abnormal(n_edges, edge_classes, _EDGE_COLS)(edge_t)
    s_n = _build_abnormal(n_nodes, node_classes, _NODE_COLS)(node_t)
    s_e0, s_n0 = s_e, s_n

    edge_update = _build_edge_update(n_edges, n_nodes)
    edge_accum = _build_edge_accum(n_edges, n_nodes, True)
    deg_accum = _build_edge_accum(n_edges, n_nodes, False)
    node_upd1 = _build_node_update(n_nodes, True)
    node_upd2 = _build_node_update(n_nodes, False)

    deg_p = deg_accum(src, dst)
    deg = None
    for it in range(NUM_ITERATIONS):
        s_e = edge_update(src, dst, s_n, s_e, gm_e_vec)
        sums = edge_accum(src, dst, s_e)
        if it == 0:
            s_n, deg = node_upd1(s_n, sums, deg_p, gm_n_s)
        else:
            s_n = node_upd2(s_n, sums, deg, gm_n_s)

    node_out = jnp.transpose(
        _build_refine(n_nodes, node_classes, _NODE_COLS)(node_t, s_n0, s_n))
    edge_out = jnp.transpose(
        _build_refine(n_edges, edge_classes, _EDGE_COLS)(edge_t, s_e0, s_e))
    return (node_out, edge_out)


# speedup vs baseline: 1.0020x; 1.0020x over previous
import functools

import jax
import jax.numpy as jnp
from jax import lax
from jax.experimental import pallas as pl
from jax.experimental.pallas import tpu as pltpu
from jax.experimental.pallas import tpu_sc as plsc

NUM_ITERATIONS = 2
GAMMA = 1.0

_NC = 2
_NS = 16
_NW = _NC * _NS
_L = 16

_EDGE_BLOCK = 2000
_ACC_BLOCK = 4000


def _npad(n_nodes):
    return ((n_nodes + _NW * 4 - 1) // (_NW * 4)) * (_NW * 4)


def _sc_mesh():
    return plsc.VectorSubcoreMesh(core_axis_name="c", subcore_axis_name="s",
                                  num_cores=_NC, num_subcores=_NS)


@functools.lru_cache(maxsize=None)
def _build_edge_update(n_edges, n_nodes):
    epw = n_edges // _NW
    block = _EDGE_BLOCK
    nblocks = epw // block

    def body(src_hbm, dst_hbm, sn_hbm, se_hbm, gm_hbm, snew_hbm,
             table_v, gm_v,
             src0_v, dst0_v, sin0_v, sout0_v, sem0,
             src1_v, dst1_v, sin1_v, sout1_v, sem1):
        wid = lax.axis_index("c") * _NS + lax.axis_index("s")
        pltpu.sync_copy(sn_hbm, table_v)
        pltpu.sync_copy(gm_hbm, gm_v)
        base0 = wid * epw
        slots = ((src0_v, dst0_v, sin0_v, sout0_v, sem0),
                 (src1_v, dst1_v, sin1_v, sout1_v, sem1))

        def copies(b, slot):
            src_v, dst_v, sin_v, _, sem = slot
            base = base0 + b * block
            return (
                pltpu.make_async_copy(src_hbm.at[pl.ds(base, block)],
                                      src_v, sem),
                pltpu.make_async_copy(dst_hbm.at[pl.ds(base, block)],
                                      dst_v, sem),
                pltpu.make_async_copy(se_hbm.at[pl.ds(base, block)],
                                      sin_v, sem))

        def start_in(b, slot):
            for c in copies(b, slot):
                c.start()

        def wait_in(b, slot):
            for c in copies(b, slot):
                c.wait()

        def compute(b, slot):
            src_v, dst_v, sin_v, sout_v, _ = slot
            gm = gm_v[...]

            def step(i, c2):
                o = i * _L
                isrc = src_v[pl.ds(o, _L)]
                idst = dst_v[pl.ds(o, _L)]
                nap = jnp.maximum(plsc.load_gather(table_v, [isrc]),
                                  plsc.load_gather(table_v, [idst]))
                f = 1.0 + nap * gm
                s = sin_v[pl.ds(o, _L)]
                sf = s * f
                sout_v[pl.ds(o, _L)] = sf / (1.0 - s + sf)
                return c2
            lax.fori_loop(0, block // _L, step, 0, unroll=8)
            base = base0 + b * block
            pltpu.sync_copy(sout_v, snew_hbm.at[pl.ds(base, block)])

        start_in(0, slots[0])

        def pair(g, carry):
            b0 = 2 * g
            start_in(b0 + 1, slots[1])
            wait_in(b0, slots[0])
            compute(b0, slots[0])

            @pl.when(g + 1 < nblocks // 2)
            def _():
                start_in(b0 + 2, slots[0])
            wait_in(b0 + 1, slots[1])
            compute(b0 + 1, slots[1])
            return carry
        lax.fori_loop(0, nblocks // 2, pair, 0)

    buf = lambda dt: pltpu.VMEM((block,), dt)
    return pl.kernel(
        body,
        out_type=jax.ShapeDtypeStruct((n_edges,), jnp.float32),
        mesh=_sc_mesh(),
        scratch_types=(pltpu.VMEM((n_nodes,), jnp.float32),
                       pltpu.VMEM((_L,), jnp.float32),
                       buf(jnp.int32), buf(jnp.int32), buf(jnp.float32),
                       buf(jnp.float32), pltpu.SemaphoreType.DMA,
                       buf(jnp.int32), buf(jnp.int32), buf(jnp.float32),
                       buf(jnp.float32), pltpu.SemaphoreType.DMA),
        compiler_params=pltpu.CompilerParams(needs_layout_passes=False))


@functools.lru_cache(maxsize=None)
def _build_edge_accum(n_edges, n_nodes, with_vals):
    npad = _npad(n_nodes)
    epw = n_edges // _NW
    block = _ACC_BLOCK
    nblocks = epw // block

    nbuf = [pltpu.VMEM((block,), jnp.int32),
            pltpu.VMEM((block,), jnp.int32)]
    if with_vals:
        nbuf.append(pltpu.VMEM((block,), jnp.float32))
    nbuf.append(pltpu.SemaphoreType.DMA)
    scratch = [pltpu.VMEM((npad,), jnp.float32)] + nbuf + nbuf

    def body(src_hbm, dst_hbm, *rest):
        nslot = 4 if with_vals else 3
        if with_vals:
            vals_hbm, out_hbm = rest[:2]
            rest = rest[2:]
        else:
            out_hbm = rest[0]
            rest = rest[1:]
        acc_v = rest[0]
        slots = (rest[1:1 + nslot], rest[1 + nslot:1 + 2 * nslot])
        wid = lax.axis_index("c") * _NS + lax.axis_index("s")

        def zstep(i, carry):
            acc_v[pl.ds(i * _L, _L)] = jnp.zeros((_L,), jnp.float32)
            return carry
        lax.fori_loop(0, npad // _L, zstep, 0, unroll=4)

        base0 = wid * epw
        ones = jnp.ones((_L,), jnp.float32)

        def copies(b, slot):
            base = base0 + b * block
            sem = slot[-1]
            cps = [pltpu.make_async_copy(src_hbm.at[pl.ds(base, block)],
                                         slot[0], sem),
                   pltpu.make_async_copy(dst_hbm.at[pl.ds(base, block)],
                                         slot[1], sem)]
            if with_vals:
                cps.append(pltpu.make_async_copy(
                    vals_hbm.at[pl.ds(base, block)], slot[2], sem))
            return cps

        def start_in(b, slot):
            for c in copies(b, slot):
                c.start()

        def wait_in(b, slot):
            for c in copies(b, slot):
                c.wait()

        def compute(slot):
            src_v, dst_v = slot[0], slot[1]

            def step(i, c2):
                o = i * _L
                v = slot[2][pl.ds(o, _L)] if with_vals else ones
                plsc.addupdate_scatter(acc_v, [src_v[pl.ds(o, _L)]], v)
                plsc.addupdate_scatter(acc_v, [dst_v[pl.ds(o, _L)]], v)
                return c2
            lax.fori_loop(0, block // _L, step, 0, unroll=4)

        start_in(0, slots[0])

        def pair(g, carry):
            b0 = 2 * g
            start_in(b0 + 1, slots[1])
            wait_in(b0, slots[0])
            compute(slots[0])

            @pl.when(g + 1 < nblocks // 2)
            def _():
                start_in(b0 + 2, slots[0])
            wait_in(b0 + 1, slots[1])
            compute(slots[1])
            return carry
        lax.fori_loop(0, nblocks // 2, pair, 0)

        pltpu.sync_copy(acc_v, out_hbm.at[wid])

    return pl.kernel(
        body,
        out_type=jax.ShapeDtypeStruct((_NW, npad), jnp.float32),
        mesh=_sc_mesh(),
        scratch_types=tuple(scratch),
        compiler_params=pltpu.CompilerParams(needs_layout_passes=False))


_EDGE_COLS = 51200
_NODE_COLS = 16384


def _abnormal_t_body(x_ref, o_ref):
    x = x_ref[...]
    m = jnp.max(x, axis=0)
    e = jnp.exp(x - m[None, :])
    z = jnp.sum(e, axis=0)
    o_ref[...] = 1.0 - e[0, :] / z


@functools.lru_cache(maxsize=None)
def _build_abnormal(cols, n_classes, block_cols):
    grid = pl.cdiv(cols, block_cols)
    return pl.pallas_call(
        _abnormal_t_body,
        grid=(grid,),
        in_specs=[pl.BlockSpec((n_classes, block_cols), lambda i: (0, i))],
        out_specs=pl.BlockSpec((block_cols,), lambda i: (i,)),
        out_shape=jax.ShapeDtypeStruct((cols,), jnp.float32),
    )


@functools.lru_cache(maxsize=None)
def _build_node_update(n_nodes, first):
    npad = _npad(n_nodes)

    def body(s_ref, sums_ref, deg_ref, gm_ref, snew_ref, *rest):
        s = s_ref[...]
        sums = jnp.sum(sums_ref[...], axis=0)[:n_nodes]
        if first:
            deg = jnp.sum(deg_ref[...], axis=0)[:n_nodes]
        else:
            deg = deg_ref[...]
        m = sums / (deg + 1e-6)
        f = 1.0 + gm_ref[0] * m
        sf = s * f
        snew_ref[...] = sf / (1.0 - s + sf)
        if first:
            rest[0][...] = deg

    deg_spec = (pl.BlockSpec((_NW, npad), lambda: (0, 0)) if first
                else pl.BlockSpec((n_nodes,), lambda: (0,)))
    if first:
        out_shape = (jax.ShapeDtypeStruct((n_nodes,), jnp.float32),
                     jax.ShapeDtypeStruct((n_nodes,), jnp.float32))
        out_specs = (pl.BlockSpec((n_nodes,), lambda: (0,)),
                     pl.BlockSpec((n_nodes,), lambda: (0,)))
    else:
        out_shape = jax.ShapeDtypeStruct((n_nodes,), jnp.float32)
        out_specs = pl.BlockSpec((n_nodes,), lambda: (0,))

    return pl.pallas_call(
        body,
        grid=(),
        in_specs=[pl.BlockSpec((n_nodes,), lambda: (0,)),
                  pl.BlockSpec((_NW, npad), lambda: (0, 0)),
                  deg_spec,
                  pl.BlockSpec(memory_space=pltpu.SMEM)],
        out_specs=out_specs,
        out_shape=out_shape,
    )


def _refine_t_body(x_ref, s0_ref, s2_ref, o_ref):
    x = x_ref[...]
    m = jnp.max(x, axis=0, keepdims=True)
    e = jnp.exp(x - m)
    z = jnp.sum(e, axis=0, keepdims=True)
    p = e / z
    s0 = s0_ref[...]
    s2 = s2_ref[...]
    r = (s2 / jnp.maximum(s0, 1e-30))[None, :]
    row = lax.broadcasted_iota(jnp.int32, x.shape, 0)
    vals = jnp.where(row == 0, (1.0 - s2)[None, :], p * r)
    o_ref[...] = jnp.log(vals + 1e-9)


@functools.lru_cache(maxsize=None)
def _build_refine(cols, n_classes, block_cols):
    grid = pl.cdiv(cols, block_cols)
    return pl.pallas_call(
        _refine_t_body,
        grid=(grid,),
        in_specs=[pl.BlockSpec((n_classes, block_cols), lambda i: (0, i)),
                  pl.BlockSpec((block_cols,), lambda i: (i,)),
                  pl.BlockSpec((block_cols,), lambda i: (i,))],
        out_specs=pl.BlockSpec((n_classes, block_cols), lambda i: (0, i)),
        out_shape=jax.ShapeDtypeStruct((n_classes, cols), jnp.float32),
    )


def kernel(node_logits, edge_logits, edge_index, node_factor_weights,
           edge_factor_weights):
    n_nodes, node_classes = node_logits.shape
    n_edges, edge_classes = edge_logits.shape

    src = edge_index[0].astype(jnp.int32)
    dst = edge_index[1].astype(jnp.int32)

    gm_e = GAMMA * jnp.mean(edge_factor_weights[1:, 1:].astype(jnp.float32))
    gm_n = GAMMA * jnp.mean(node_factor_weights[1:, 1:].astype(jnp.float32))
    gm_e_vec = jnp.full((_L,), gm_e, jnp.float32)
    gm_n_s = jnp.reshape(gm_n, (1,))

    edge_t = jnp.transpose(edge_logits)
    node_t = jnp.transpose(node_logits)
    s_e = _build_abnormal(n_edges, edge_classes, _EDGE_COLS)(edge_t)
    s_n = _build_abnormal(n_nodes, node_classes, _NODE_COLS)(node_t)
    s_e0, s_n0 = s_e, s_n

    edge_update = _build_edge_update(n_edges, n_nodes)
    edge_accum = _build_edge_accum(n_edges, n_nodes, True)
    deg_accum = _build_edge_accum(n_edges, n_nodes, False)
    node_upd1 = _build_node_update(n_nodes, True)
    node_upd2 = _build_node_update(n_nodes, False)

    deg_p = deg_accum(src, dst)
    deg = None
    for it in range(NUM_ITERATIONS):
        s_e = edge_update(src, dst, s_n, s_e, gm_e_vec)
        sums = edge_accum(src, dst, s_e)
        if it == 0:
            s_n, deg = node_upd1(s_n, sums, deg_p, gm_n_s)
        else:
            s_n = node_upd2(s_n, sums, deg, gm_n_s)

    node_out = jnp.transpose(
        _build_refine(n_nodes, node_classes, _NODE_COLS)(node_t, s_n0, s_n))
    edge_out = jnp.transpose(
        _build_refine(n_edges, edge_classes, _EDGE_COLS)(edge_t, s_e0, s_e))
    return (node_out, edge_out)
